# baseline (speedup 1.0000x reference)
import functools

import jax
import jax.numpy as jnp
from jax import lax
from jax.experimental import pallas as pl
from jax.experimental.pallas import tpu as pltpu
from jax.experimental.pallas import tpu_sc as plsc

F = 26
V = 100000
D = 32
B = 16384

NC = 2
NS = 16
NW = NC * NS
BPW = B // NW
CHUNK = 128
NB = BPW // CHUNK
NCH = F * NB


def _sc_body(tab_hbm, idx_hbm, out_hbm, idx_v, stage_v, acc_v, sem0, sem1):
    wid = lax.axis_index("s") * NC + lax.axis_index("c")
    base_b = wid * BPW

    pltpu.sync_copy(idx_hbm.at[wid], idx_v)

    zeros16 = jnp.zeros((16,), jnp.float32)

    def zbody(r, c):
        acc_v[r, pl.ds(0, 16)] = zeros16
        acc_v[r, pl.ds(16, 16)] = zeros16
        return c

    lax.fori_loop(0, BPW, zbody, 0)

    sems = (sem0, sem1)

    def issue(g, b):
        pltpu.async_copy(tab_hbm.at[idx_v.at[g]], stage_v.at[b], sems[b])

    def wait(g, b):
        pltpu.make_async_copy(tab_hbm.at[idx_v.at[g]], stage_v.at[b],
                              sems[b]).wait()

    issue(0, 0)
    issue(1, 1)

    def outer(g0, c):
        for b in range(2):
            g = g0 + b
            wait(g, b)

            base_r = lax.rem(g, NB) * CHUNK

            def abody(r, c2):
                plsc.addupdate(acc_v.at[base_r + r, pl.ds(0, 16)],
                               stage_v[b, r, pl.ds(0, 16)])
                plsc.addupdate(acc_v.at[base_r + r, pl.ds(16, 16)],
                               stage_v[b, r, pl.ds(16, 16)])
                return c2

            lax.fori_loop(0, CHUNK, abody, 0, unroll=4)

            @pl.when(g + 2 < NCH)
            def _():
                issue(g + 2, b)
        return c

    lax.fori_loop(0, NCH // 2, lambda i, c: outer(i * 2, c), 0)

    pltpu.sync_copy(acc_v, out_hbm.at[pl.ds(base_b, BPW)])


@jax.jit
def _sc_lookup(tab_flat, idx):
    mesh = plsc.VectorSubcoreMesh(core_axis_name="c", subcore_axis_name="s",
                                  num_cores=NC, num_subcores=NS)
    return pl.kernel(
        _sc_body,
        out_type=jax.ShapeDtypeStruct((B, D), jnp.float32),
        mesh=mesh,
        scratch_types=[
            pltpu.VMEM((NCH, CHUNK), jnp.int32),
            pltpu.VMEM((2, CHUNK, D), jnp.float32),
            pltpu.VMEM((BPW, D), jnp.float32),
            pltpu.SemaphoreType.DMA,
            pltpu.SemaphoreType.DMA,
        ],
        compiler_params=pltpu.CompilerParams(use_tc_tiling_on_sc=False),
    )(tab_flat, idx)


def kernel(x, tables):
    tab_flat = tables.reshape(F * V, D)
    flat = x + (jnp.arange(F, dtype=jnp.int32) * V)[None, :]
    idx = flat.reshape(NW, NB, CHUNK, F).transpose(0, 3, 1, 2)
    idx = idx.reshape(NW, NCH, CHUNK)
    return _sc_lookup(tab_flat, idx)
